# baseline (speedup 1.0000x reference)
import functools

import jax
import jax.numpy as jnp
from jax import lax
from jax.experimental import pallas as pl
from jax.experimental.pallas import tpu as pltpu
from jax.experimental.pallas import tpu_sc as plsc

B = 512
N = 45000
E = 100000
NC = 2
NS = 16
NW = NC * NS
ROWS_PER_W = B // NW
CHUNK = 10000
N_CHUNKS = E // CHUNK
VECS = CHUNK // 16


def _sc_kernel(probs_hbm, child_hbm, parent_hbm, out_hbm,
               row0_v, row1_v, ci_v, pi_v, out_v):
    wid = lax.axis_index("s") * NC + lax.axis_index("c")
    row_base = wid * ROWS_PER_W

    def pair_body(rp, acc):
        r0 = row_base + 2 * rp
        pltpu.sync_copy(probs_hbm.at[r0], row0_v)
        pltpu.sync_copy(probs_hbm.at[r0 + 1], row1_v)

        def chunk_body(ch, acc):
            off = ch * CHUNK
            pltpu.sync_copy(child_hbm.at[pl.ds(off, CHUNK)], ci_v)
            pltpu.sync_copy(parent_hbm.at[pl.ds(off, CHUNK)], pi_v)

            def vec_body(i, acc):
                ci = ci_v[pl.ds(i * 16, 16)]
                pi = pi_v[pl.ds(i * 16, 16)]
                c0 = plsc.load_gather(row0_v, [ci])
                p0 = plsc.load_gather(row0_v, [pi])
                c1 = plsc.load_gather(row1_v, [ci])
                p1 = plsc.load_gather(row1_v, [pi])
                zero = jnp.zeros((16,), jnp.float32)
                acc = acc + jnp.maximum(c0 - p0, zero)
                acc = acc + jnp.maximum(c1 - p1, zero)
                return acc

            return lax.fori_loop(0, VECS, vec_body, acc)

        return lax.fori_loop(0, N_CHUNKS, chunk_body, acc)

    acc = lax.fori_loop(0, ROWS_PER_W // 2, pair_body,
                        jnp.zeros((16,), jnp.float32))
    out_v[...] = acc
    pltpu.sync_copy(out_v, out_hbm.at[wid])


@jax.jit
def _hierarchical_loss(probs, child, parent):
    mesh = plsc.VectorSubcoreMesh(core_axis_name="c", subcore_axis_name="s",
                                  num_cores=NC, num_subcores=NS)
    partials = pl.kernel(
        _sc_kernel,
        out_type=jax.ShapeDtypeStruct((NW, 16), jnp.float32),
        mesh=mesh,
        compiler_params=pltpu.CompilerParams(needs_layout_passes=False),
        scratch_types=[
            pltpu.VMEM((N,), jnp.float32),
            pltpu.VMEM((N,), jnp.float32),
            pltpu.VMEM((CHUNK,), jnp.int32),
            pltpu.VMEM((CHUNK,), jnp.int32),
            pltpu.VMEM((16,), jnp.float32),
        ],
    )(probs, child, parent)
    return jnp.sum(partials)


def kernel(probs, edge_index):
    child = edge_index[0].astype(jnp.int32)
    parent = edge_index[1].astype(jnp.int32)
    return _hierarchical_loss(probs, child, parent)
